# baseline (speedup 1.0000x reference)
import math

import jax
import jax.numpy as jnp
import numpy as np
from jax.experimental import pallas as pl

_SEGS = 16
_TOP_K = 8
_NHEAD = 4


def _fused_kernel(a_ref, v_ref, q_ref, wq_ref, wk_ref, bq_ref, bk_ref,
                  wfc_ref, bfc_ref, oa_ref, ov_ref, idx_ref):
    T, C = a_ref.shape[1], a_ref.shape[2]
    clip_len = T // _SEGS
    hd = C // _NHEAD

    a = a_ref[0]
    v = v_ref[0]
    a_clip = a.reshape(_SEGS, clip_len, C).mean(axis=1)
    v_clip = v.reshape(_SEGS, clip_len, C).mean(axis=1)

    fusion = jnp.tanh(jnp.concatenate([a_clip, v_clip], axis=-1))
    fusion = jax.lax.dot_general(
        fusion, wfc_ref[...], (((1,), (1,)), ((), ())),
        preferred_element_type=jnp.float32) + bfc_ref[...]

    q = jax.lax.dot_general(
        q_ref[0], wq_ref[...], (((1,), (1,)), ((), ())),
        preferred_element_type=jnp.float32) + bq_ref[...]
    k = jax.lax.dot_general(
        fusion, wk_ref[...], (((1,), (1,)), ((), ())),
        preferred_element_type=jnp.float32) + bk_ref[...]

    qh = q.astype(jnp.bfloat16).astype(jnp.float32).reshape(_NHEAD, hd)
    kh = k.astype(jnp.bfloat16).astype(jnp.float32).reshape(_SEGS, _NHEAD, hd)
    scores = (kh * qh[None]).sum(axis=-1) / np.float32(math.sqrt(hd))

    m = scores.max(axis=0, keepdims=True)
    e = jnp.exp(scores - m)
    attn = e / e.sum(axis=0, keepdims=True)
    w_col = attn.mean(axis=1, keepdims=True)
    w_row = w_col.T

    ii = jax.lax.broadcasted_iota(jnp.int32, (_SEGS, _SEGS), 0)
    jj = jax.lax.broadcasted_iota(jnp.int32, (_SEGS, _SEGS), 1)
    beats = (w_row > w_col) | ((w_row == w_col) & (jj > ii))
    count_col = jnp.sum(beats.astype(jnp.int32), axis=1, keepdims=True)
    count_row = (_SEGS - 1) - jnp.sum(beats.astype(jnp.int32), axis=0,
                                      keepdims=True)
    in_top_col = count_col < _TOP_K
    in_top_row = count_row < _TOP_K
    pos_col = jnp.sum((in_top_row & (jj < ii)).astype(jnp.int32), axis=1,
                      keepdims=True)
    r_row = jax.lax.broadcasted_iota(jnp.int32, (1, _TOP_K), 1)
    oh = in_top_col & (pos_col == r_row)
    i8 = jax.lax.broadcasted_iota(jnp.int32, (_SEGS, _TOP_K), 0)
    idx_row = jnp.sum(jnp.where(oh, i8, 0), axis=0, keepdims=True)
    idx_ref[0] = idx_row

    for kk in range(_TOP_K):
        seg = idx_row[0, kk]
        off = seg * clip_len
        oa_ref[0, kk * clip_len:(kk + 1) * clip_len, :] = \
            a_ref[0, pl.ds(off, clip_len), :]
        ov_ref[0, kk * clip_len:(kk + 1) * clip_len, :] = \
            v_ref[0, pl.ds(off, clip_len), :]


def kernel(audio_input, visual_input, qst_input, in_proj_w, in_proj_b,
           clip_fc_w, clip_fc_b):
    B, T, C = audio_input.shape
    clip_len = T // _SEGS
    wq = in_proj_w[:C]
    wk = in_proj_w[C:2 * C]
    bq = in_proj_b[:C].reshape(1, C)
    bk = in_proj_b[C:2 * C].reshape(1, C)
    bfc = clip_fc_b.reshape(1, C)

    out_shape = [
        jax.ShapeDtypeStruct((B, _TOP_K * clip_len, C), jnp.float32),
        jax.ShapeDtypeStruct((B, _TOP_K * clip_len, C), jnp.float32),
        jax.ShapeDtypeStruct((B, 1, _TOP_K), jnp.int32),
    ]
    in_specs = [
        pl.BlockSpec((1, T, C), lambda b: (b, 0, 0)),
        pl.BlockSpec((1, T, C), lambda b: (b, 0, 0)),
        pl.BlockSpec((1, 1, C), lambda b: (b, 0, 0)),
        pl.BlockSpec((C, C), lambda b: (0, 0)),
        pl.BlockSpec((C, C), lambda b: (0, 0)),
        pl.BlockSpec((1, C), lambda b: (0, 0)),
        pl.BlockSpec((1, C), lambda b: (0, 0)),
        pl.BlockSpec((C, 2 * C), lambda b: (0, 0)),
        pl.BlockSpec((1, C), lambda b: (0, 0)),
    ]
    out_specs = [
        pl.BlockSpec((1, _TOP_K * clip_len, C), lambda b: (b, 0, 0)),
        pl.BlockSpec((1, _TOP_K * clip_len, C), lambda b: (b, 0, 0)),
        pl.BlockSpec((1, 1, _TOP_K), lambda b: (b, 0, 0)),
    ]
    oa, ov, idx = pl.pallas_call(
        _fused_kernel,
        grid=(B,),
        in_specs=in_specs,
        out_specs=out_specs,
        out_shape=out_shape,
    )(audio_input, visual_input, qst_input.reshape(B, 1, C), wq, wk, bq, bk,
      clip_fc_w, bfc)
    return (oa, ov, idx)


# speedup vs baseline: 1.0162x; 1.0162x over previous
import math

import jax
import jax.numpy as jnp
import numpy as np
from jax.experimental import pallas as pl

_SEGS = 16
_TOP_K = 8
_NHEAD = 4


def _fused_kernel(a_ref, v_ref, q_ref, wk_ref, bk_ref,
                  wfc_ref, bfc_ref, oa_ref, ov_ref, idx_ref):
    T, C = a_ref.shape[1], a_ref.shape[2]
    clip_len = T // _SEGS
    hd = C // _NHEAD

    a = a_ref[0]
    v = v_ref[0]
    a_clip = a.reshape(_SEGS, clip_len, C).mean(axis=1)
    v_clip = v.reshape(_SEGS, clip_len, C).mean(axis=1)

    fusion = jnp.tanh(jnp.concatenate([a_clip, v_clip], axis=-1))
    fusion = jax.lax.dot_general(
        fusion, wfc_ref[...], (((1,), (1,)), ((), ())),
        preferred_element_type=jnp.float32) + bfc_ref[...]

    q = q_ref[0]
    k = jax.lax.dot_general(
        fusion, wk_ref[...], (((1,), (1,)), ((), ())),
        preferred_element_type=jnp.float32) + bk_ref[...]

    qh = q.astype(jnp.bfloat16).astype(jnp.float32).reshape(_NHEAD, hd)
    kh = k.astype(jnp.bfloat16).astype(jnp.float32).reshape(_SEGS, _NHEAD, hd)
    scores = (kh * qh[None]).sum(axis=-1) / np.float32(math.sqrt(hd))

    m = scores.max(axis=0, keepdims=True)
    e = jnp.exp(scores - m)
    attn = e / e.sum(axis=0, keepdims=True)
    w_col = attn.mean(axis=1, keepdims=True)
    w_row = w_col.T

    ii = jax.lax.broadcasted_iota(jnp.int32, (_SEGS, _SEGS), 0)
    jj = jax.lax.broadcasted_iota(jnp.int32, (_SEGS, _SEGS), 1)
    beats = (w_row > w_col) | ((w_row == w_col) & (jj > ii))
    count_col = jnp.sum(beats.astype(jnp.int32), axis=1, keepdims=True)
    count_row = (_SEGS - 1) - jnp.sum(beats.astype(jnp.int32), axis=0,
                                      keepdims=True)
    in_top_col = count_col < _TOP_K
    in_top_row = count_row < _TOP_K
    pos_col = jnp.sum((in_top_row & (jj < ii)).astype(jnp.int32), axis=1,
                      keepdims=True)
    r_row = jax.lax.broadcasted_iota(jnp.int32, (1, _TOP_K), 1)
    oh = in_top_col & (pos_col == r_row)
    i8 = jax.lax.broadcasted_iota(jnp.int32, (_SEGS, _TOP_K), 0)
    idx_row = jnp.sum(jnp.where(oh, i8, 0), axis=0, keepdims=True)
    idx_ref[0] = idx_row

    for kk in range(_TOP_K):
        seg = idx_row[0, kk]
        off = seg * clip_len
        oa_ref[0, kk * clip_len:(kk + 1) * clip_len, :] = \
            a_ref[0, pl.ds(off, clip_len), :]
        ov_ref[0, kk * clip_len:(kk + 1) * clip_len, :] = \
            v_ref[0, pl.ds(off, clip_len), :]


def kernel(audio_input, visual_input, qst_input, in_proj_w, in_proj_b,
           clip_fc_w, clip_fc_b):
    B, T, C = audio_input.shape
    clip_len = T // _SEGS
    wq = in_proj_w[:C]
    wk = in_proj_w[C:2 * C]
    bk = in_proj_b[C:2 * C].reshape(1, C)
    bfc = clip_fc_b.reshape(1, C)
    q_all = (qst_input @ wq.T + in_proj_b[:C]).reshape(B, 1, C)

    out_shape = [
        jax.ShapeDtypeStruct((B, _TOP_K * clip_len, C), jnp.float32),
        jax.ShapeDtypeStruct((B, _TOP_K * clip_len, C), jnp.float32),
        jax.ShapeDtypeStruct((B, 1, _TOP_K), jnp.int32),
    ]
    in_specs = [
        pl.BlockSpec((1, T, C), lambda b: (b, 0, 0)),
        pl.BlockSpec((1, T, C), lambda b: (b, 0, 0)),
        pl.BlockSpec((1, 1, C), lambda b: (b, 0, 0)),
        pl.BlockSpec((C, C), lambda b: (0, 0)),
        pl.BlockSpec((1, C), lambda b: (0, 0)),
        pl.BlockSpec((C, 2 * C), lambda b: (0, 0)),
        pl.BlockSpec((1, C), lambda b: (0, 0)),
    ]
    out_specs = [
        pl.BlockSpec((1, _TOP_K * clip_len, C), lambda b: (b, 0, 0)),
        pl.BlockSpec((1, _TOP_K * clip_len, C), lambda b: (b, 0, 0)),
        pl.BlockSpec((1, 1, _TOP_K), lambda b: (b, 0, 0)),
    ]
    oa, ov, idx = pl.pallas_call(
        _fused_kernel,
        grid=(B,),
        in_specs=in_specs,
        out_specs=out_specs,
        out_shape=out_shape,
    )(audio_input, visual_input, q_all, wk, bk, clip_fc_w, bfc)
    return (oa, ov, idx)
